# baseline (speedup 1.0000x reference)
import functools

import jax
import jax.numpy as jnp
from jax import lax
from jax.experimental import pallas as pl
from jax.experimental.pallas import tpu as pltpu
from jax.experimental.pallas import tpu_sc as plsc

N = 100000
Z = 100000
S = 4
U = 32
D = 3
F = S * U
B = 400
NBLK = N // B
NW = 32
PAIRS = 4
BT = 5000


def _sc_body(x0_hbm, i0_hbm, gfwd_hbm,
             idx0, idx1, g0, g1,
             si0, si1, sg0, sg1, so0, so1):
    wid = lax.axis_index("s") * 2 + lax.axis_index("c")
    idx = (idx0, idx1)
    gg = (g0, g1)
    si = (si0, si1)
    sg = (sg0, sg1)
    so = (so0, so1)

    def fire_idx(t, p):
        blk = wid + t * NW

        @pl.when(blk < NBLK)
        def _():
            pltpu.async_copy(i0_hbm.at[pl.ds(blk * B, B)], idx[p], si[p])

    def wait_idx(t, p):
        blk = wid + t * NW

        @pl.when(blk < NBLK)
        def _():
            pltpu.make_async_copy(i0_hbm.at[pl.ds(blk * B, B)], idx[p],
                                  si[p]).wait()

    def fire_gather(t, b):
        blk = wid + t * NW

        @pl.when(blk < NBLK)
        def _():
            pltpu.async_copy(x0_hbm.at[idx[b]], gg[b], sg[b])

    def wait_gather(t, b):
        blk = wid + t * NW

        @pl.when(blk < NBLK)
        def _():
            pltpu.make_async_copy(x0_hbm.at[idx[b]], gg[b], sg[b]).wait()

    def fire_fwd(t, b):
        blk = wid + t * NW

        @pl.when(blk < NBLK)
        def _():
            pltpu.async_copy(gg[b], gfwd_hbm.at[pl.ds(blk * B, B)], so[b])

    def wait_fwd(t, b):
        blk = wid + t * NW

        @pl.when((t >= 0) & (blk < NBLK))
        def _():
            pltpu.make_async_copy(gg[b], gfwd_hbm.at[pl.ds(blk * B, B)],
                                  so[b]).wait()

    fire_idx(0, 0)
    fire_idx(1, 1)
    wait_idx(0, 0)
    fire_gather(0, 0)

    def pair(i, _):
        for b in range(2):
            t = 2 * i + b
            wait_gather(t, b)
            wait_idx(t + 1, 1 - b)
            fire_gather(t + 1, 1 - b)
            fire_idx(t + 2, b)
            wait_fwd(t - 2, b)
            fire_fwd(t, b)
        return _

    lax.fori_loop(0, PAIRS, pair, None)
    wait_fwd(2 * PAIRS - 2, 0)
    wait_fwd(2 * PAIRS - 1, 1)


def _tc_body(g_ref, x_ref, w_ref, o_ref):
    g = g_ref[...].astype(jnp.bfloat16)
    x = x_ref[...]
    m = [jnp.dot(g, w_ref[d], preferred_element_type=jnp.float32)
         for d in range(D)]
    r2 = m[D - 1]
    for d in range(D - 2, -1, -1):
        r2 = r2 * x + m[d]
    o_ref[...] = r2 * x


@jax.jit
def _run(x0, i0, x1, w):
    mesh = plsc.VectorSubcoreMesh(core_axis_name="c", subcore_axis_name="s")
    sc_fn = functools.partial(
        pl.kernel,
        mesh=mesh,
        out_type=jax.ShapeDtypeStruct((N, F), jnp.float32),
        scratch_types=[
            pltpu.VMEM((B,), jnp.int32),
            pltpu.VMEM((B,), jnp.int32),
            pltpu.VMEM((B, F), jnp.float32),
            pltpu.VMEM((B, F), jnp.float32),
            pltpu.SemaphoreType.DMA,
            pltpu.SemaphoreType.DMA,
            pltpu.SemaphoreType.DMA,
            pltpu.SemaphoreType.DMA,
            pltpu.SemaphoreType.DMA,
            pltpu.SemaphoreType.DMA,
        ],
    )(_sc_body)
    g_fwd = sc_fn(x0, i0)

    out = pl.pallas_call(
        _tc_body,
        grid=(N // BT,),
        in_specs=[
            pl.BlockSpec((BT, F), lambda i: (i, 0)),
            pl.BlockSpec((BT, F), lambda i: (i, 0)),
            pl.BlockSpec((D, F, F), lambda i: (0, 0, 0)),
        ],
        out_specs=pl.BlockSpec((BT, F), lambda i: (i, 0)),
        out_shape=jax.ShapeDtypeStruct((N, F), jnp.float32),
        compiler_params=pltpu.CompilerParams(
            dimension_semantics=("parallel",)),
    )(g_fwd, x1, w)
    return out


def kernel(x0, i0, x1, C):
    i0 = i0.astype(jnp.int32)
    w = jnp.einsum('dos,uv->dsuov', C, jnp.eye(U, dtype=jnp.float32))
    w = w.reshape(D, F, F).astype(jnp.bfloat16)
    return _run(x0, i0, x1, w)


# speedup vs baseline: 1.0510x; 1.0510x over previous
import functools

import jax
import jax.numpy as jnp
from jax import lax
from jax.experimental import pallas as pl
from jax.experimental.pallas import tpu as pltpu
from jax.experimental.pallas import tpu_sc as plsc

N = 100000
Z = 100000
S = 4
U = 32
D = 3
F = S * U
B = 400
NBLK = N // B
NW = 32
PAIRS = 4
BT = 4000


def _sc_body(x0_hbm, i0_hbm, gfwd_hbm,
             idx0, idx1, g0, g1,
             si0, si1, sg0, sg1, so0, so1):
    wid = lax.axis_index("s") * 2 + lax.axis_index("c")
    idx = (idx0, idx1)
    gg = (g0, g1)
    si = (si0, si1)
    sg = (sg0, sg1)
    so = (so0, so1)

    def fire_idx(t, p):
        blk = wid + t * NW

        @pl.when(blk < NBLK)
        def _():
            pltpu.async_copy(i0_hbm.at[pl.ds(blk * B, B)], idx[p], si[p])

    def wait_idx(t, p):
        blk = wid + t * NW

        @pl.when(blk < NBLK)
        def _():
            pltpu.make_async_copy(i0_hbm.at[pl.ds(blk * B, B)], idx[p],
                                  si[p]).wait()

    def fire_gather(t, b):
        blk = wid + t * NW

        @pl.when(blk < NBLK)
        def _():
            pltpu.async_copy(x0_hbm.at[idx[b]], gg[b], sg[b])

    def wait_gather(t, b):
        blk = wid + t * NW

        @pl.when(blk < NBLK)
        def _():
            pltpu.make_async_copy(x0_hbm.at[idx[b]], gg[b], sg[b]).wait()

    def fire_fwd(t, b):
        blk = wid + t * NW

        @pl.when(blk < NBLK)
        def _():
            pltpu.async_copy(gg[b], gfwd_hbm.at[pl.ds(blk * B, B)], so[b])

    def wait_fwd(t, b):
        blk = wid + t * NW

        @pl.when((t >= 0) & (blk < NBLK))
        def _():
            pltpu.make_async_copy(gg[b], gfwd_hbm.at[pl.ds(blk * B, B)],
                                  so[b]).wait()

    fire_idx(0, 0)
    fire_idx(1, 1)
    wait_idx(0, 0)
    fire_gather(0, 0)

    def pair(i, _):
        for b in range(2):
            t = 2 * i + b
            wait_gather(t, b)
            wait_idx(t + 1, 1 - b)
            fire_gather(t + 1, 1 - b)
            fire_idx(t + 2, b)
            wait_fwd(t - 2, b)
            fire_fwd(t, b)
        return _

    lax.fori_loop(0, PAIRS, pair, None)
    wait_fwd(2 * PAIRS - 2, 0)
    wait_fwd(2 * PAIRS - 1, 1)


def _tc_body(g_ref, x_ref, w_ref, o_ref):
    g = g_ref[...].astype(jnp.bfloat16)
    x = x_ref[...]
    m = [jnp.dot(g, w_ref[d], preferred_element_type=jnp.float32)
         for d in range(D)]
    r2 = m[D - 1]
    for d in range(D - 2, -1, -1):
        r2 = r2 * x + m[d]
    o_ref[...] = r2 * x


@jax.jit
def _run(x0, i0, x1, w):
    mesh = plsc.VectorSubcoreMesh(core_axis_name="c", subcore_axis_name="s")
    sc_fn = functools.partial(
        pl.kernel,
        mesh=mesh,
        out_type=jax.ShapeDtypeStruct((N, F), jnp.float32),
        scratch_types=[
            pltpu.VMEM((B,), jnp.int32),
            pltpu.VMEM((B,), jnp.int32),
            pltpu.VMEM((B, F), jnp.float32),
            pltpu.VMEM((B, F), jnp.float32),
            pltpu.SemaphoreType.DMA,
            pltpu.SemaphoreType.DMA,
            pltpu.SemaphoreType.DMA,
            pltpu.SemaphoreType.DMA,
            pltpu.SemaphoreType.DMA,
            pltpu.SemaphoreType.DMA,
        ],
    )(_sc_body)
    g_fwd = sc_fn(x0, i0)

    out = pl.pallas_call(
        _tc_body,
        grid=(N // BT,),
        in_specs=[
            pl.BlockSpec((BT, F), lambda i: (i, 0)),
            pl.BlockSpec((BT, F), lambda i: (i, 0)),
            pl.BlockSpec((D, F, F), lambda i: (0, 0, 0)),
        ],
        out_specs=pl.BlockSpec((BT, F), lambda i: (i, 0)),
        out_shape=jax.ShapeDtypeStruct((N, F), jnp.float32),
        compiler_params=pltpu.CompilerParams(
            dimension_semantics=("parallel",)),
    )(g_fwd, x1, w)
    return out


def kernel(x0, i0, x1, C):
    i0 = i0.astype(jnp.int32)
    w = jnp.einsum('dos,uv->dsuov', C, jnp.eye(U, dtype=jnp.float32))
    w = w.reshape(D, F, F).astype(jnp.bfloat16)
    return _run(x0, i0, x1, w)
